# baseline (speedup 1.0000x reference)
import functools

import jax
import jax.numpy as jnp
from jax import lax
from jax.experimental import pallas as pl
from jax.experimental.pallas import tpu as pltpu
from jax.experimental.pallas import tpu_sc as plsc

LOCAL = 12500
EMB = 128
NZ = 512
NROWS = LOCAL + NZ
BATCH = 4096
SEQ = 200
TOKENS = BATCH * SEQ

NC, NS, L = 2, 16, 16
NW = NC * NS
TPW = TOKENS // NW
CHUNK = 64
IBLK = 1024
NBLK = TPW // IBLK
CPB = IBLK // CHUNK
NCHUNK = TPW // CHUNK


def _sc_body(ids_hbm, table_hbm, out_hbm, ids_v, rows_v, table_sp, gsem,
             wsem, isem):
    sid = lax.axis_index("s")
    wid = sid * NC + lax.axis_index("c")
    base = wid * TPW

    @pl.when(sid == 0)
    def _():
        pltpu.sync_copy(table_hbm, table_sp)

    def drain_ids(slot):
        pltpu.make_async_copy(
            ids_hbm.at[pl.ds(0, IBLK)], ids_v.at[pl.ds(0, IBLK)],
            isem.at[slot]).wait()

    def issue_gather(islot, cc, b):
        pltpu.async_copy(
            table_sp.at[ids_v.at[pl.ds(islot * IBLK + cc * CHUNK, CHUNK)]],
            rows_v.at[pl.ds(b * CHUNK, CHUNK)], gsem.at[b])

    def drain_gather(b):
        pltpu.make_async_copy(
            table_sp.at[pl.ds(0, CHUNK)], rows_v.at[pl.ds(0, CHUNK)],
            gsem.at[b]).wait()

    def issue_write(c, b):
        pltpu.async_copy(
            rows_v.at[pl.ds(b * CHUNK, CHUNK)],
            out_hbm.at[pl.ds(base + c * CHUNK, CHUNK)], wsem.at[b])

    def drain_write(b):
        pltpu.make_async_copy(
            rows_v.at[pl.ds(0, CHUNK)], out_hbm.at[pl.ds(0, CHUNK)],
            wsem.at[b]).wait()

    for blk0 in range(2):
        pltpu.async_copy(
            ids_hbm.at[pl.ds(base + blk0 * IBLK, IBLK)],
            ids_v.at[pl.ds(blk0 * IBLK, IBLK)], isem.at[blk0])
    plsc.subcore_barrier()

    def clamp(islot, cc):
        for j in range(CHUNK // L):
            o = islot * IBLK + cc * CHUNK + j * L
            v = ids_v[pl.ds(o, L)]
            zrow = LOCAL + (v & (NZ - 1))
            ids_v[pl.ds(o, L)] = jnp.where(v < LOCAL, v, zrow)

    def block_loop(blk, carry):
        for slot in range(2):
            @pl.when(blk % 2 == slot)
            def _():
                drain_ids(slot)

                for cc in range(CPB):
                    b = cc % 2
                    c = blk * CPB + cc

                    @pl.when(c >= 2)
                    def _():
                        drain_write(b)

                    clamp(slot, cc)
                    issue_gather(slot, cc, b)

                    @pl.when(c >= 1)
                    def _():
                        drain_gather(1 - b)
                        issue_write(c - 1, 1 - b)

                @pl.when(blk + 2 < NBLK)
                def _():
                    pltpu.async_copy(
                        ids_hbm.at[pl.ds(base + (blk + 2) * IBLK, IBLK)],
                        ids_v.at[pl.ds(slot * IBLK, IBLK)], isem.at[slot])
        return carry

    lax.fori_loop(0, NBLK, block_loop, 0)

    last = NCHUNK - 1
    bl = last % 2
    drain_gather(bl)
    issue_write(last, bl)
    drain_write(1 - bl)
    drain_write(bl)


@functools.partial(
    pl.kernel,
    mesh=plsc.VectorSubcoreMesh(core_axis_name="c", subcore_axis_name="s"),
    out_type=jax.ShapeDtypeStruct((TOKENS, EMB), jnp.float32),
    scratch_types=[
        pltpu.VMEM((2 * IBLK,), jnp.int32),
        pltpu.VMEM((2 * CHUNK, EMB), jnp.float32),
        pltpu.VMEM_SHARED((NROWS, EMB), jnp.float32),
        pltpu.SemaphoreType.DMA((2,)),
        pltpu.SemaphoreType.DMA((2,)),
        pltpu.SemaphoreType.DMA((2,)),
    ],
)
def _sc_lookup(ids_hbm, table_hbm, out_hbm, ids_v, rows_v, table_sp, gsem,
               wsem, isem):
    _sc_body(ids_hbm, table_hbm, out_hbm, ids_v, rows_v, table_sp, gsem,
             wsem, isem)


def kernel(input_ids, weight):
    ids = input_ids.reshape(TOKENS)
    table = jnp.concatenate([weight, jnp.zeros((NZ, EMB), weight.dtype)],
                            axis=0)
    out = _sc_lookup(ids, table)
    return out.reshape(BATCH, SEQ, EMB)


# speedup vs baseline: 1.0022x; 1.0022x over previous
import functools

import jax
import jax.numpy as jnp
from jax import lax
from jax.experimental import pallas as pl
from jax.experimental.pallas import tpu as pltpu
from jax.experimental.pallas import tpu_sc as plsc

LOCAL = 12500
EMB = 128
NZ = 64
NROWS = LOCAL + NZ
BATCH = 4096
SEQ = 200
TOKENS = BATCH * SEQ

NC, NS, L = 2, 16, 16
NW = NC * NS
TPW = TOKENS // NW
CHUNK = 64
IBLK = 1024
NBLK = TPW // IBLK
CPB = IBLK // CHUNK
NCHUNK = TPW // CHUNK


def _sc_body(ids_hbm, table_hbm, out_hbm, ids_v, rows_v, table_sp, gsem,
             wsem, isem):
    sid = lax.axis_index("s")
    wid = sid * NC + lax.axis_index("c")
    base = wid * TPW

    @pl.when(sid == 0)
    def _():
        pltpu.sync_copy(table_hbm, table_sp)

    def drain_ids(slot):
        pltpu.make_async_copy(
            ids_hbm.at[pl.ds(0, IBLK)], ids_v.at[pl.ds(0, IBLK)],
            isem.at[slot]).wait()

    def issue_gather(islot, cc, b):
        pltpu.async_copy(
            table_sp.at[ids_v.at[pl.ds(islot * IBLK + cc * CHUNK, CHUNK)]],
            rows_v.at[pl.ds(b * CHUNK, CHUNK)], gsem.at[b])

    def drain_gather(b):
        pltpu.make_async_copy(
            table_sp.at[pl.ds(0, CHUNK)], rows_v.at[pl.ds(0, CHUNK)],
            gsem.at[b]).wait()

    def issue_write(c, b):
        pltpu.async_copy(
            rows_v.at[pl.ds(b * CHUNK, CHUNK)],
            out_hbm.at[pl.ds(base + c * CHUNK, CHUNK)], wsem.at[b])

    def drain_write(b):
        pltpu.make_async_copy(
            rows_v.at[pl.ds(0, CHUNK)], out_hbm.at[pl.ds(0, CHUNK)],
            wsem.at[b]).wait()

    for blk0 in range(2):
        pltpu.async_copy(
            ids_hbm.at[pl.ds(base + blk0 * IBLK, IBLK)],
            ids_v.at[pl.ds(blk0 * IBLK, IBLK)], isem.at[blk0])
    plsc.subcore_barrier()

    def clamp(islot, cc):
        for j in range(CHUNK // L):
            o = islot * IBLK + cc * CHUNK + j * L
            v = ids_v[pl.ds(o, L)]
            zrow = LOCAL + (v & (NZ - 1))
            ids_v[pl.ds(o, L)] = jnp.where(v < LOCAL, v, zrow)

    def block_loop(blk, carry):
        for slot in range(2):
            @pl.when(blk % 2 == slot)
            def _():
                drain_ids(slot)

                for cc in range(CPB):
                    b = cc % 2
                    c = blk * CPB + cc

                    @pl.when(c >= 2)
                    def _():
                        drain_write(b)

                    clamp(slot, cc)
                    issue_gather(slot, cc, b)

                    @pl.when(c >= 1)
                    def _():
                        drain_gather(1 - b)
                        issue_write(c - 1, 1 - b)

                @pl.when(blk + 2 < NBLK)
                def _():
                    pltpu.async_copy(
                        ids_hbm.at[pl.ds(base + (blk + 2) * IBLK, IBLK)],
                        ids_v.at[pl.ds(slot * IBLK, IBLK)], isem.at[slot])
        return carry

    lax.fori_loop(0, NBLK, block_loop, 0)

    last = NCHUNK - 1
    bl = last % 2
    drain_gather(bl)
    issue_write(last, bl)
    drain_write(1 - bl)
    drain_write(bl)


@functools.partial(
    pl.kernel,
    mesh=plsc.VectorSubcoreMesh(core_axis_name="c", subcore_axis_name="s"),
    out_type=jax.ShapeDtypeStruct((TOKENS, EMB), jnp.float32),
    scratch_types=[
        pltpu.VMEM((2 * IBLK,), jnp.int32),
        pltpu.VMEM((2 * CHUNK, EMB), jnp.float32),
        pltpu.VMEM_SHARED((NROWS, EMB), jnp.float32),
        pltpu.SemaphoreType.DMA((2,)),
        pltpu.SemaphoreType.DMA((2,)),
        pltpu.SemaphoreType.DMA((2,)),
    ],
)
def _sc_lookup(ids_hbm, table_hbm, out_hbm, ids_v, rows_v, table_sp, gsem,
               wsem, isem):
    _sc_body(ids_hbm, table_hbm, out_hbm, ids_v, rows_v, table_sp, gsem,
             wsem, isem)


def kernel(input_ids, weight):
    ids = input_ids.reshape(TOKENS)
    table = jnp.concatenate([weight, jnp.zeros((NZ, EMB), weight.dtype)],
                            axis=0)
    out = _sc_lookup(ids, table)
    return out.reshape(BATCH, SEQ, EMB)


# speedup vs baseline: 1.0693x; 1.0669x over previous
import functools

import jax
import jax.numpy as jnp
from jax import lax
from jax.experimental import pallas as pl
from jax.experimental.pallas import tpu as pltpu
from jax.experimental.pallas import tpu_sc as plsc

LOCAL = 12500
EMB = 128
NZ = 64
NROWS = LOCAL + NZ
BATCH = 4096
SEQ = 200
TOKENS = BATCH * SEQ

NC, NS, L = 2, 16, 16
NW = NC * NS
TPW = TOKENS // NW
CHUNK = 64
IBLK = 1024
NBLK = TPW // IBLK
CPB = IBLK // CHUNK
NCHUNK = TPW // CHUNK


def _sc_body(ids_hbm, table_hbm, out_hbm, ids_v, rows_v, table_sp, gsem,
             wsem, isem):
    sid = lax.axis_index("s")
    wid = sid * NC + lax.axis_index("c")
    base = wid * TPW

    @pl.when(sid == 0)
    def _():
        pltpu.sync_copy(table_hbm, table_sp)

    def drain_ids(slot):
        pltpu.make_async_copy(
            ids_hbm.at[pl.ds(0, IBLK)], ids_v.at[pl.ds(0, IBLK)],
            isem.at[slot]).wait()

    def issue_gather(islot, cc, b):
        pltpu.async_copy(
            table_sp.at[ids_v.at[pl.ds(islot * IBLK + cc * CHUNK, CHUNK)]],
            rows_v.at[pl.ds(pl.multiple_of(b * CHUNK, 8), CHUNK)], gsem.at[b])

    def drain_gather(b):
        pltpu.make_async_copy(
            table_sp.at[pl.ds(0, CHUNK)], rows_v.at[pl.ds(0, CHUNK)],
            gsem.at[b]).wait()

    def issue_write(c, b):
        pltpu.async_copy(
            rows_v.at[pl.ds(pl.multiple_of(b * CHUNK, 8), CHUNK)],
            out_hbm.at[pl.ds(base + c * CHUNK, CHUNK)], wsem.at[b])

    def drain_write(b):
        pltpu.make_async_copy(
            rows_v.at[pl.ds(0, CHUNK)], out_hbm.at[pl.ds(0, CHUNK)],
            wsem.at[b]).wait()

    for blk0 in range(2):
        pltpu.async_copy(
            ids_hbm.at[pl.ds(base + blk0 * IBLK, IBLK)],
            ids_v.at[pl.ds(blk0 * IBLK, IBLK)], isem.at[blk0])
    plsc.subcore_barrier()

    def clamp(islot, cc):
        for j in range(CHUNK // L):
            o = islot * IBLK + cc * CHUNK + j * L
            v = ids_v[pl.ds(o, L)]
            zrow = LOCAL + (v & (NZ - 1))
            ids_v[pl.ds(o, L)] = jnp.where(v < LOCAL, v, zrow)

    def block_loop(blk, carry):
        for slot in range(2):
            @pl.when(blk % 2 == slot)
            def _():
                drain_ids(slot)

                for cc in range(CPB):
                    c = blk * CPB + cc
                    b = lax.rem(c, 3)
                    bp = lax.rem(c + 2, 3)

                    @pl.when(c >= 3)
                    def _():
                        drain_write(b)

                    clamp(slot, cc)
                    issue_gather(slot, cc, b)

                    @pl.when(c >= 1)
                    def _():
                        drain_gather(bp)
                        issue_write(c - 1, bp)

                @pl.when(blk + 2 < NBLK)
                def _():
                    pltpu.async_copy(
                        ids_hbm.at[pl.ds(base + (blk + 2) * IBLK, IBLK)],
                        ids_v.at[pl.ds(slot * IBLK, IBLK)], isem.at[slot])
        return carry

    lax.fori_loop(0, NBLK, block_loop, 0)

    last = NCHUNK - 1
    bl = last % 3
    drain_gather(bl)
    issue_write(last, bl)
    for k in range(3):
        drain_write((last - 2 + k) % 3)


@functools.partial(
    pl.kernel,
    mesh=plsc.VectorSubcoreMesh(core_axis_name="c", subcore_axis_name="s"),
    out_type=jax.ShapeDtypeStruct((TOKENS, EMB), jnp.float32),
    scratch_types=[
        pltpu.VMEM((2 * IBLK,), jnp.int32),
        pltpu.VMEM((3 * CHUNK, EMB), jnp.float32),
        pltpu.VMEM_SHARED((NROWS, EMB), jnp.float32),
        pltpu.SemaphoreType.DMA((3,)),
        pltpu.SemaphoreType.DMA((3,)),
        pltpu.SemaphoreType.DMA((2,)),
    ],
)
def _sc_lookup(ids_hbm, table_hbm, out_hbm, ids_v, rows_v, table_sp, gsem,
               wsem, isem):
    _sc_body(ids_hbm, table_hbm, out_hbm, ids_v, rows_v, table_sp, gsem,
             wsem, isem)


def kernel(input_ids, weight):
    ids = input_ids.reshape(TOKENS)
    table = jnp.concatenate([weight, jnp.zeros((NZ, EMB), weight.dtype)],
                            axis=0)
    out = _sc_lookup(ids, table)
    return out.reshape(BATCH, SEQ, EMB)
